# baseline (speedup 1.0000x reference)
import functools

import jax
import jax.numpy as jnp
from jax import lax
from jax.experimental import pallas as pl
from jax.experimental.pallas import tpu as pltpu
from jax.experimental.pallas import tpu_sc as plsc

B, N, DIM = 4, 4096, 1024
K = 1024
LH = DIM // 2
HH = DIM * 4
RT = B * N
PK = B * K
ZPAD = 256
BT1 = 2048
BT2 = 512
NLANES = 16
NV = N // NLANES
NTILES = 32
_SQRT_HALF = 0.7071067811865476


def _gelu(h):
    return 0.5 * h * (1.0 + lax.erf(h * _SQRT_HALF))


def _rms(x, gamma_row, scale):
    ss = jnp.sum(x * x, axis=1, keepdims=True)
    return x * (scale / jnp.maximum(jnp.sqrt(ss), 1e-12)) * gamma_row


def _light_body(x_ref, rt_ref, g_ref, w1_ref, b1_ref, w2_ref, b2_ref,
                out_ref, sc_ref):
    x = x_ref[...]
    sc_ref[...] = jnp.dot(x, rt_ref[...], preferred_element_type=jnp.float32)
    normed = _rms(x, g_ref[...], DIM ** 0.5)
    h = jnp.dot(normed.astype(jnp.bfloat16), w1_ref[...],
                preferred_element_type=jnp.float32)
    h = _gelu(h + b1_ref[...])
    out_ref[...] = jnp.dot(h.astype(jnp.bfloat16), w2_ref[...],
                           preferred_element_type=jnp.float32) + b2_ref[...]


def _light_call(xf, rtp, g2, w1, b12, w2, b22):
    g = RT // BT1
    return pl.pallas_call(
        _light_body,
        grid=(g,),
        in_specs=[
            pl.BlockSpec((BT1, DIM), lambda i: (i, 0)),
            pl.BlockSpec((DIM, 128), lambda i: (0, 0)),
            pl.BlockSpec((1, DIM), lambda i: (0, 0)),
            pl.BlockSpec((DIM, LH), lambda i: (0, 0)),
            pl.BlockSpec((1, LH), lambda i: (0, 0)),
            pl.BlockSpec((LH, DIM), lambda i: (0, 0)),
            pl.BlockSpec((1, DIM), lambda i: (0, 0)),
        ],
        out_specs=[
            pl.BlockSpec((BT1, DIM), lambda i: (i, 0)),
            pl.BlockSpec((BT1, 128), lambda i: (i, 0)),
        ],
        out_shape=[
            jax.ShapeDtypeStruct((RT, DIM), jnp.float32),
            jax.ShapeDtypeStruct((RT, 128), jnp.float32),
        ],
    )(xf, rtp, g2, w1, b12, w2, b22)


def _heavy_body(p_ref, g_ref, w1_ref, b1_ref, w2_ref, b2_ref, out_ref):
    x = p_ref[...]
    normed = _rms(x, g_ref[...], DIM ** 0.5)
    h = jnp.dot(normed.astype(jnp.bfloat16), w1_ref[...],
                preferred_element_type=jnp.float32)
    h = _gelu(h + b1_ref[...])
    out_ref[...] = jnp.dot(h.astype(jnp.bfloat16), w2_ref[...],
                           preferred_element_type=jnp.float32) + b2_ref[...]


def _heavy_call(packed, g2, w1, b12, w2, b22):
    g = PK // BT2
    return pl.pallas_call(
        _heavy_body,
        grid=(g,),
        in_specs=[
            pl.BlockSpec((BT2, DIM), lambda i: (i, 0)),
            pl.BlockSpec((1, DIM), lambda i: (0, 0)),
            pl.BlockSpec((DIM, HH), lambda i: (0, 0)),
            pl.BlockSpec((1, HH), lambda i: (0, 0)),
            pl.BlockSpec((HH, DIM), lambda i: (0, 0)),
            pl.BlockSpec((1, DIM), lambda i: (0, 0)),
        ],
        out_specs=pl.BlockSpec((BT2, DIM), lambda i: (i, 0)),
        out_shape=jax.ShapeDtypeStruct((PK, DIM), jnp.float32),
    )(packed, g2, w1, b12, w2, b22)


def _wid():
    return lax.axis_index("s") * 2 + lax.axis_index("c")


_SC_PARAMS = pltpu.CompilerParams(needs_layout_passes=False)


def _make_topk(mesh):
    return functools.partial(
        pl.kernel,
        mesh=mesh,
        compiler_params=_SC_PARAMS,
        out_type=jax.ShapeDtypeStruct((PK,), jnp.int32),
        scratch_types=[pltpu.VMEM((N,), jnp.float32),
                       pltpu.VMEM((N,), jnp.uint32),
                       pltpu.VMEM((K,), jnp.int32)],
    )(_topk_body)


def _topk_body(scores_hbm, idx_hbm, sv, kv, idxv):
    wid = _wid()

    @pl.when(wid < B)
    def _():
        b = wid
        pltpu.sync_copy(scores_hbm.at[pl.ds(b * N, N)], sv)

        def conv(j, _):
            bits = lax.bitcast_convert_type(sv[pl.ds(j * NLANES, NLANES)],
                                            jnp.uint32)
            key = jnp.where(bits >> 31 != 0, ~bits,
                            bits | jnp.uint32(0x80000000))
            kv[pl.ds(j * NLANES, NLANES)] = key
            return 0

        lax.fori_loop(0, NV, conv, 0, unroll=4)

        def bs(i, carry):
            lo, bitv = carry
            cand = lo | bitv

            def cnt(j, acc):
                m = kv[pl.ds(j * NLANES, NLANES)] >= cand
                return acc + jnp.where(m, 1, 0)

            acc = lax.fori_loop(0, NV, cnt, jnp.zeros((NLANES,), jnp.int32),
                                unroll=8)
            npass = plsc.all_reduce_population_count(plsc.cumsum(acc) >= K)
            lo = jnp.where(npass > 0, cand, lo)
            return lo, bitv >> jnp.uint32(1)

        t, _unused = lax.fori_loop(
            0, 32, bs,
            (jnp.zeros((NLANES,), jnp.uint32),
             jnp.full((NLANES,), 0x80000000, jnp.uint32)))

        def pass_a(j, cur):
            k16 = kv[pl.ds(j * NLANES, NLANES)]
            m = k16 > t
            pos = cur + plsc.cumsum(jnp.where(m, 1, 0)) - 1
            tok = j * NLANES + lax.iota(jnp.int32, NLANES)
            plsc.store_scatter(idxv, [pos], b * N + tok, mask=m)
            return cur + plsc.all_reduce_population_count(m)

        c1 = lax.fori_loop(0, NV, pass_a, jnp.zeros((NLANES,), jnp.int32),
                           unroll=4)

        def pass_b(j, cur):
            k16 = kv[pl.ds(j * NLANES, NLANES)]
            m = k16 == t
            pos = cur + plsc.cumsum(jnp.where(m, 1, 0)) - 1
            mv = m & (pos < K)
            tok = j * NLANES + lax.iota(jnp.int32, NLANES)
            plsc.store_scatter(idxv, [pos], b * N + tok, mask=mv)
            return cur + plsc.all_reduce_population_count(m)

        lax.fori_loop(0, NV, pass_b, c1, unroll=4)

        pltpu.sync_copy(idxv, idx_hbm.at[pl.ds(b * K, K)])


_PGT = PK // NTILES
_GCH = 32
_GNCH = _PGT // _GCH


def _make_gather(mesh):
    return functools.partial(
        pl.kernel,
        mesh=mesh,
        out_type=jax.ShapeDtypeStruct((PK, DIM), jnp.float32),
        scratch_types=[pltpu.VMEM((_PGT,), jnp.int32),
                       pltpu.VMEM((_GCH, DIM), jnp.float32),
                       pltpu.VMEM((_GCH, DIM), jnp.float32),
                       pltpu.SemaphoreType.DMA,
                       pltpu.SemaphoreType.DMA,
                       pltpu.SemaphoreType.DMA,
                       pltpu.SemaphoreType.DMA],
    )(_gather_body)


def _gather_body(xf_hbm, idx_hbm, packed_hbm, idxv, rows0, rows1,
                 sg0, sg1, sw0, sw1):
    base = _wid() * _PGT
    sets = ((rows0, sg0, sw0), (rows1, sg1, sw1))
    pend = [None, None]
    wb = [None, None]
    pltpu.sync_copy(idx_hbm.at[pl.ds(base, _PGT)], idxv)
    for c in range(_GNCH + 1):
        if c < _GNCH:
            rows, sg, sw = sets[c % 2]
            if wb[c % 2] is not None:
                wb[c % 2].wait()
            pend[c % 2] = pltpu.async_copy(
                xf_hbm.at[idxv.at[pl.ds(c * _GCH, _GCH)]], rows, sg)
        if c >= 1:
            d = c - 1
            rows, sg, sw = sets[d % 2]
            pend[d % 2].wait()
            wb[d % 2] = pltpu.async_copy(
                rows, packed_hbm.at[pl.ds(base + d * _GCH, _GCH)], sw)
    wb[0].wait()
    wb[1].wait()


_RPP = PK // NTILES
_RCH = 16
_RNCH = _RPP // _RCH
_RVPC = _RCH * DIM // NLANES


def _make_rmw(mesh):
    return functools.partial(
        pl.kernel,
        mesh=mesh,
        out_type=(),
        scratch_types=[pltpu.VMEM((_RCH, DIM), jnp.float32),
                       pltpu.VMEM((_RCH, DIM), jnp.float32),
                       pltpu.VMEM((_RCH,), jnp.int32),
                       pltpu.VMEM((_RCH, DIM), jnp.float32),
                       pltpu.VMEM((_RCH, DIM), jnp.float32),
                       pltpu.VMEM((_RCH,), jnp.int32),
                       pltpu.SemaphoreType.DMA,
                       pltpu.SemaphoreType.DMA,
                       pltpu.SemaphoreType.DMA,
                       pltpu.SemaphoreType.DMA,
                       pltpu.SemaphoreType.DMA,
                       pltpu.SemaphoreType.DMA],
    )(_rmw_body)


def _rmw_body(out_hbm, routed_hbm, idx_hbm,
              gbuf0, rbuf0, idxc0, gbuf1, rbuf1, idxc1,
              sem_g0, sem_r0, sem_w0, sem_g1, sem_r1, sem_w1):
    base = _wid() * _RPP
    sets = ((gbuf0, rbuf0, idxc0, sem_g0, sem_r0, sem_w0),
            (gbuf1, rbuf1, idxc1, sem_g1, sem_r1, sem_w1))
    pend = [None, None]
    wb = [None, None]

    for c in range(_RNCH + 1):
        if c < _RNCH:
            gb, rb, ic, sg, sr, sw = sets[c % 2]
            st = base + c * _RCH
            if wb[c % 2] is not None:
                wb[c % 2].wait()
            pltpu.sync_copy(idx_hbm.at[pl.ds(st, _RCH)], ic)
            h_g = pltpu.async_copy(out_hbm.at[ic], gb, sg)
            h_r = pltpu.async_copy(routed_hbm.at[pl.ds(st, _RCH)], rb, sr)
            pend[c % 2] = (h_g, h_r)
        if c >= 1:
            d = c - 1
            gb, rb, ic, sg, sr, sw = sets[d % 2]
            h_g, h_r = pend[d % 2]
            h_g.wait()
            h_r.wait()

            def add16(t, _):
                r = t // (DIM // NLANES)
                jj = t % (DIM // NLANES)
                s = (r, pl.ds(jj * NLANES, NLANES))
                gb[s] = gb[s] + rb[s]
                return 0

            lax.fori_loop(0, _RVPC, add16, 0, unroll=8)
            wb[d % 2] = pltpu.async_copy(gb, out_hbm.at[ic], sw)
    wb[0].wait()
    wb[1].wait()


@functools.cache
def _sc_kernels():
    mesh = plsc.VectorSubcoreMesh(core_axis_name="c", subcore_axis_name="s")
    return _make_topk(mesh), _make_gather(mesh), _make_rmw(mesh)


def kernel(x, routing_token, gamma_light, w1_light, b1_light, w2_light,
           b2_light, gamma_heavy, w1_heavy, b1_heavy, w2_heavy, b2_heavy):
    _topk, _gather, _rmw = _sc_kernels()
    xf = x.reshape(RT, DIM)
    rtp = jnp.zeros((DIM, 128), jnp.float32).at[:, 0].set(routing_token)
    light_out, s128 = _light_call(
        xf, rtp, gamma_light.reshape(1, DIM),
        w1_light.astype(jnp.bfloat16), b1_light.reshape(1, LH),
        w2_light.astype(jnp.bfloat16), b2_light.reshape(1, DIM))
    scores = s128[:, 0]
    idx = _topk(scores)
    packed = _gather(xf, idx)
    routed = _heavy_call(
        packed, gamma_heavy.reshape(1, DIM), w1_heavy.astype(jnp.bfloat16),
        b1_heavy.reshape(1, HH), w2_heavy.astype(jnp.bfloat16),
        b2_heavy.reshape(1, DIM))
    out_ref = jax.new_ref(light_out)
    _rmw(out_ref, routed, idx)
    return out_ref[...].reshape(B, N, DIM)


# speedup vs baseline: 1.0021x; 1.0021x over previous
import functools

import jax
import jax.numpy as jnp
from jax import lax
from jax.experimental import pallas as pl
from jax.experimental.pallas import tpu as pltpu
from jax.experimental.pallas import tpu_sc as plsc

B, N, DIM = 4, 4096, 1024
K = 1024
LH = DIM // 2
HH = DIM * 4
RT = B * N
PK = B * K
BT1 = 1024
BT2 = 512
NLANES = 16
NV = N // NLANES
NTILES = 32
_SQRT_HALF = 0.7071067811865476


def _gelu(h):
    return 0.5 * h * (1.0 + lax.erf(h * _SQRT_HALF))


def _rms(x, gamma_row, scale):
    ss = jnp.sum(x * x, axis=1, keepdims=True)
    return x * (scale / jnp.maximum(jnp.sqrt(ss), 1e-12)) * gamma_row


def _light_body(x_ref, rt_ref, g_ref, w1_ref, b1_ref, w2_ref, b2_ref,
                out_ref, sc_ref):
    x = x_ref[...]
    sc_ref[...] = jnp.dot(x, rt_ref[...], preferred_element_type=jnp.float32)
    normed = _rms(x, g_ref[...], DIM ** 0.5)
    h = jnp.dot(normed.astype(jnp.bfloat16), w1_ref[...],
                preferred_element_type=jnp.float32)
    h = _gelu(h + b1_ref[...])
    out_ref[...] = jnp.dot(h.astype(jnp.bfloat16), w2_ref[...],
                           preferred_element_type=jnp.float32) + b2_ref[...]


def _light_call(xf, rtp, g2, w1, b12, w2, b22):
    g = RT // BT1
    return pl.pallas_call(
        _light_body,
        grid=(g,),
        in_specs=[
            pl.BlockSpec((BT1, DIM), lambda i: (i, 0)),
            pl.BlockSpec((DIM, 128), lambda i: (0, 0)),
            pl.BlockSpec((1, DIM), lambda i: (0, 0)),
            pl.BlockSpec((DIM, LH), lambda i: (0, 0)),
            pl.BlockSpec((1, LH), lambda i: (0, 0)),
            pl.BlockSpec((LH, DIM), lambda i: (0, 0)),
            pl.BlockSpec((1, DIM), lambda i: (0, 0)),
        ],
        out_specs=[
            pl.BlockSpec((BT1, DIM), lambda i: (i, 0)),
            pl.BlockSpec((BT1, 128), lambda i: (i, 0)),
        ],
        out_shape=[
            jax.ShapeDtypeStruct((RT, DIM), jnp.float32),
            jax.ShapeDtypeStruct((RT, 128), jnp.float32),
        ],
    )(xf, rtp, g2, w1, b12, w2, b22)


def _heavy_body(p_ref, g_ref, w1_ref, b1_ref, w2_ref, b2_ref, out_ref):
    x = p_ref[...]
    normed = _rms(x, g_ref[...], DIM ** 0.5)
    h = jnp.dot(normed.astype(jnp.bfloat16), w1_ref[...],
                preferred_element_type=jnp.float32)
    h = _gelu(h + b1_ref[...])
    out_ref[...] = jnp.dot(h.astype(jnp.bfloat16), w2_ref[...],
                           preferred_element_type=jnp.float32) + b2_ref[...]


def _heavy_call(packed, g2, w1, b12, w2, b22):
    g = PK // BT2
    return pl.pallas_call(
        _heavy_body,
        grid=(g,),
        in_specs=[
            pl.BlockSpec((BT2, DIM), lambda i: (i, 0)),
            pl.BlockSpec((1, DIM), lambda i: (0, 0)),
            pl.BlockSpec((DIM, HH), lambda i: (0, 0)),
            pl.BlockSpec((1, HH), lambda i: (0, 0)),
            pl.BlockSpec((HH, DIM), lambda i: (0, 0)),
            pl.BlockSpec((1, DIM), lambda i: (0, 0)),
        ],
        out_specs=pl.BlockSpec((BT2, DIM), lambda i: (i, 0)),
        out_shape=jax.ShapeDtypeStruct((PK, DIM), jnp.float32),
    )(packed, g2, w1, b12, w2, b22)


def _wid():
    return lax.axis_index("s") * 2 + lax.axis_index("c")


_SC_PARAMS = pltpu.CompilerParams(needs_layout_passes=False)


def _make_topk(mesh):
    return functools.partial(
        pl.kernel,
        mesh=mesh,
        compiler_params=_SC_PARAMS,
        out_type=jax.ShapeDtypeStruct((PK,), jnp.int32),
        scratch_types=[pltpu.VMEM((N,), jnp.float32),
                       pltpu.VMEM((N,), jnp.uint32),
                       pltpu.VMEM((K,), jnp.int32)],
    )(_topk_body)


def _topk_body(scores_hbm, idx_hbm, sv, kv, idxv):
    wid = _wid()

    @pl.when(wid < B)
    def _():
        b = wid
        pltpu.sync_copy(scores_hbm.at[pl.ds(b * N, N)], sv)

        def conv(j, _):
            bits = lax.bitcast_convert_type(sv[pl.ds(j * NLANES, NLANES)],
                                            jnp.uint32)
            key = jnp.where(bits >> 31 != 0, ~bits,
                            bits | jnp.uint32(0x80000000))
            kv[pl.ds(j * NLANES, NLANES)] = key
            return 0

        lax.fori_loop(0, NV, conv, 0, unroll=4)

        def bs(i, carry):
            lo, bitv = carry
            cand = lo | bitv

            def cnt(j, acc):
                m = kv[pl.ds(j * NLANES, NLANES)] >= cand
                return acc + jnp.where(m, 1, 0)

            acc = lax.fori_loop(0, NV, cnt, jnp.zeros((NLANES,), jnp.int32),
                                unroll=8)
            npass = plsc.all_reduce_population_count(plsc.cumsum(acc) >= K)
            lo = jnp.where(npass > 0, cand, lo)
            return lo, bitv >> jnp.uint32(1)

        t, _unused = lax.fori_loop(
            0, 32, bs,
            (jnp.zeros((NLANES,), jnp.uint32),
             jnp.full((NLANES,), 0x80000000, jnp.uint32)))

        def pass_a(j, cur):
            k16 = kv[pl.ds(j * NLANES, NLANES)]
            m = k16 > t
            pos = cur + plsc.cumsum(jnp.where(m, 1, 0)) - 1
            tok = j * NLANES + lax.iota(jnp.int32, NLANES)
            plsc.store_scatter(idxv, [pos], b * N + tok, mask=m)
            return cur + plsc.all_reduce_population_count(m)

        c1 = lax.fori_loop(0, NV, pass_a, jnp.zeros((NLANES,), jnp.int32),
                           unroll=4)

        def pass_b(j, cur):
            k16 = kv[pl.ds(j * NLANES, NLANES)]
            m = k16 == t
            pos = cur + plsc.cumsum(jnp.where(m, 1, 0)) - 1
            mv = m & (pos < K)
            tok = j * NLANES + lax.iota(jnp.int32, NLANES)
            plsc.store_scatter(idxv, [pos], b * N + tok, mask=mv)
            return cur + plsc.all_reduce_population_count(m)

        lax.fori_loop(0, NV, pass_b, c1, unroll=4)

        pltpu.sync_copy(idxv, idx_hbm.at[pl.ds(b * K, K)])


_PGT = PK // NTILES
_GCH = 32
_GNCH = _PGT // _GCH


def _make_gather(mesh):
    return functools.partial(
        pl.kernel,
        mesh=mesh,
        out_type=jax.ShapeDtypeStruct((PK, DIM), jnp.float32),
        scratch_types=[pltpu.VMEM((_PGT,), jnp.int32),
                       pltpu.VMEM((_GCH, DIM), jnp.float32),
                       pltpu.VMEM((_GCH, DIM), jnp.float32),
                       pltpu.SemaphoreType.DMA,
                       pltpu.SemaphoreType.DMA,
                       pltpu.SemaphoreType.DMA,
                       pltpu.SemaphoreType.DMA],
    )(_gather_body)


def _gather_body(xf_hbm, idx_hbm, packed_hbm, idxv, rows0, rows1,
                 sg0, sg1, sw0, sw1):
    base = _wid() * _PGT
    sets = ((rows0, sg0, sw0), (rows1, sg1, sw1))
    pend = [None, None]
    wb = [None, None]
    pltpu.sync_copy(idx_hbm.at[pl.ds(base, _PGT)], idxv)
    for c in range(_GNCH + 1):
        if c < _GNCH:
            rows, sg, sw = sets[c % 2]
            if wb[c % 2] is not None:
                wb[c % 2].wait()
            pend[c % 2] = pltpu.async_copy(
                xf_hbm.at[idxv.at[pl.ds(c * _GCH, _GCH)]], rows, sg)
        if c >= 1:
            d = c - 1
            rows, sg, sw = sets[d % 2]
            pend[d % 2].wait()
            wb[d % 2] = pltpu.async_copy(
                rows, packed_hbm.at[pl.ds(base + d * _GCH, _GCH)], sw)
    wb[0].wait()
    wb[1].wait()


_RPP = PK // NTILES
_RCH = 16
_RNCH = _RPP // _RCH
_RVPC = _RCH * DIM // NLANES


def _make_rmw(mesh):
    return functools.partial(
        pl.kernel,
        mesh=mesh,
        out_type=(),
        scratch_types=[pltpu.VMEM((_RCH, DIM), jnp.float32),
                       pltpu.VMEM((_RCH, DIM), jnp.float32),
                       pltpu.VMEM((_RCH,), jnp.int32),
                       pltpu.VMEM((_RCH, DIM), jnp.float32),
                       pltpu.VMEM((_RCH, DIM), jnp.float32),
                       pltpu.VMEM((_RCH,), jnp.int32),
                       pltpu.SemaphoreType.DMA,
                       pltpu.SemaphoreType.DMA,
                       pltpu.SemaphoreType.DMA,
                       pltpu.SemaphoreType.DMA,
                       pltpu.SemaphoreType.DMA,
                       pltpu.SemaphoreType.DMA],
    )(_rmw_body)


def _rmw_body(out_hbm, routed_hbm, idx_hbm,
              gbuf0, rbuf0, idxc0, gbuf1, rbuf1, idxc1,
              sem_g0, sem_r0, sem_w0, sem_g1, sem_r1, sem_w1):
    base = _wid() * _RPP
    sets = ((gbuf0, rbuf0, idxc0, sem_g0, sem_r0, sem_w0),
            (gbuf1, rbuf1, idxc1, sem_g1, sem_r1, sem_w1))
    pend = [None, None]
    wb = [None, None]

    for c in range(_RNCH + 1):
        if c < _RNCH:
            gb, rb, ic, sg, sr, sw = sets[c % 2]
            st = base + c * _RCH
            if wb[c % 2] is not None:
                wb[c % 2].wait()
            pltpu.sync_copy(idx_hbm.at[pl.ds(st, _RCH)], ic)
            h_g = pltpu.async_copy(out_hbm.at[ic], gb, sg)
            h_r = pltpu.async_copy(routed_hbm.at[pl.ds(st, _RCH)], rb, sr)
            pend[c % 2] = (h_g, h_r)
        if c >= 1:
            d = c - 1
            gb, rb, ic, sg, sr, sw = sets[d % 2]
            h_g, h_r = pend[d % 2]
            h_g.wait()
            h_r.wait()

            def add16(t, _):
                r = t // (DIM // NLANES)
                jj = t % (DIM // NLANES)
                s = (r, pl.ds(jj * NLANES, NLANES))
                gb[s] = gb[s] + rb[s]
                return 0

            lax.fori_loop(0, _RVPC, add16, 0, unroll=8)
            wb[d % 2] = pltpu.async_copy(gb, out_hbm.at[ic], sw)
    wb[0].wait()
    wb[1].wait()


@functools.cache
def _sc_kernels():
    mesh = plsc.VectorSubcoreMesh(core_axis_name="c", subcore_axis_name="s")
    return _make_topk(mesh), _make_gather(mesh), _make_rmw(mesh)


def kernel(x, routing_token, gamma_light, w1_light, b1_light, w2_light,
           b2_light, gamma_heavy, w1_heavy, b1_heavy, w2_heavy, b2_heavy):
    _topk, _gather, _rmw = _sc_kernels()
    xf = x.reshape(RT, DIM)
    rtp = jnp.zeros((DIM, 128), jnp.float32).at[:, 0].set(routing_token)
    light_out, s128 = _light_call(
        xf, rtp, gamma_light.reshape(1, DIM),
        w1_light.astype(jnp.bfloat16), b1_light.reshape(1, LH),
        w2_light.astype(jnp.bfloat16), b2_light.reshape(1, DIM))
    scores = s128[:, 0]
    idx = _topk(scores)
    packed = _gather(xf, idx)
    routed = _heavy_call(
        packed, gamma_heavy.reshape(1, DIM), w1_heavy.astype(jnp.bfloat16),
        b1_heavy.reshape(1, HH), w2_heavy.astype(jnp.bfloat16),
        b2_heavy.reshape(1, DIM))
    out_ref = jax.new_ref(light_out)
    _rmw(out_ref, routed, idx)
    return out_ref[...].reshape(B, N, DIM)
